# baseline (speedup 1.0000x reference)
import functools

import jax
import jax.numpy as jnp
from jax import lax
from jax.experimental import pallas as pl
from jax.experimental.pallas import tpu as pltpu
from jax.experimental.pallas import tpu_sc as plsc

V = 1000
VM = 896
VT = V - VM
VP = 1024
T = 50
TM = 48
BB = 8
NC = 2
NS = 16
NW = NC * NS


def _table_body(e_ref, wt_ref, b_ref, main_ref, tail_ref):
    res = (
        jnp.dot(e_ref[...], wt_ref[...],
                preferred_element_type=jnp.float32,
                precision=lax.Precision.HIGHEST)
        + b_ref[...]
    )
    main_ref[...] = res[:, :VM]
    tail_ref[...] = res[:, VM:V]


def _make_tables(embed_weight, wt, bias2d):
    return pl.pallas_call(
        _table_body,
        out_shape=(
            jax.ShapeDtypeStruct((V, VM), jnp.float32),
            jax.ShapeDtypeStruct((V, VT), jnp.float32),
        ),
    )(embed_weight, wt, bias2d)


def _tail_body(tok_ref, ttail_ref, out_ref):
    tt = ttail_ref[...]
    tok = tok_ref[...]
    iot = lax.broadcasted_iota(jnp.int32, (T, V), 1)
    for r in range(BB):
        onehot = (tok[r][:, None] == iot).astype(jnp.float32)
        out_ref[r] = jnp.dot(onehot, tt,
                             preferred_element_type=jnp.float32,
                             precision=lax.Precision.HIGHEST)


def _make_tail(token_ids_i32, ttail, B):
    return pl.pallas_call(
        _tail_body,
        grid=(B // BB,),
        in_specs=[
            pl.BlockSpec((BB, T), lambda i: (i, 0)),
            pl.BlockSpec((V, VT), lambda i: (0, 0)),
        ],
        out_specs=pl.BlockSpec((BB, T, VT), lambda i: (i, 0, 0)),
        out_shape=jax.ShapeDtypeStruct((B, T, VT), jnp.float32),
    )(token_ids_i32, ttail)


def _rowtail_body(tok_ref, tmain_ref, out_ref):
    tm = tmain_ref[...]
    tok = tok_ref[...]
    iot = lax.broadcasted_iota(jnp.int32, (T - TM, V), 1)
    for r in range(BB):
        onehot = (tok[r][TM:T][:, None] == iot).astype(jnp.float32)
        out_ref[r] = jnp.dot(onehot, tm,
                             preferred_element_type=jnp.float32,
                             precision=lax.Precision.HIGHEST)


def _make_rowtail(token_ids_i32, tmain, B):
    return pl.pallas_call(
        _rowtail_body,
        grid=(B // BB,),
        in_specs=[
            pl.BlockSpec((BB, T), lambda i: (i, 0)),
            pl.BlockSpec((V, VM), lambda i: (0, 0)),
        ],
        out_specs=pl.BlockSpec((BB, T - TM, VM), lambda i: (i, 0, 0)),
        out_shape=jax.ShapeDtypeStruct((B, T - TM, VM), jnp.float32),
    )(token_ids_i32, tmain)


def _gather_body(nchunk, tmain_hbm, tail_hbm, rowtail_hbm, idx_hbm, out_hbm,
                 idx_v, rm0, rm1, bt0, bt1, br0, br1,
                 semm0, semm1, semt0, semt1, semr0, semr1):
    wid = lax.axis_index("s") * NC + lax.axis_index("c")

    pltpu.sync_copy(idx_hbm.at[pl.ds(wid * nchunk * TM, nchunk * TM)],
                    idx_v)

    def fire(j, rm, bt, br, semm, semt, semr):
        b = wid * nchunk + j
        ids = idx_v.at[pl.ds(j * TM, TM)]
        pltpu.async_copy(tmain_hbm.at[ids], rm, semm)
        pltpu.async_copy(tail_hbm.at[b], bt, semt)
        pltpu.async_copy(rowtail_hbm.at[b], br, semr)

    def drain_write(j, rm, bt, br, semm, semt, semr):
        ids = idx_v.at[pl.ds(0, TM)]
        b = wid * nchunk + j
        pltpu.make_async_copy(tmain_hbm.at[ids], rm, semm).wait()
        pltpu.sync_copy(rm, out_hbm.at[b, pl.ds(0, TM), pl.ds(0, VM)])
        pltpu.make_async_copy(tail_hbm.at[b], bt, semt).wait()
        pltpu.sync_copy(bt, out_hbm.at[b, :, pl.ds(VM, VT)])
        pltpu.make_async_copy(rowtail_hbm.at[b], br, semr).wait()
        pltpu.sync_copy(br, out_hbm.at[b, pl.ds(TM, T - TM), pl.ds(0, VM)])

    fire(0, rm0, bt0, br0, semm0, semt0, semr0)
    fire(1, rm1, bt1, br1, semm1, semt1, semr1)

    def pair_step(p, carry):
        j = 2 * p
        drain_write(j, rm0, bt0, br0, semm0, semt0, semr0)
        fire(j + 2, rm0, bt0, br0, semm0, semt0, semr0)
        drain_write(j + 1, rm1, bt1, br1, semm1, semt1, semr1)
        fire(j + 3, rm1, bt1, br1, semm1, semt1, semr1)
        return carry

    lax.fori_loop(0, nchunk // 2 - 1, pair_step, 0)
    drain_write(nchunk - 2, rm0, bt0, br0, semm0, semt0, semr0)
    drain_write(nchunk - 1, rm1, bt1, br1, semm1, semt1, semr1)


def _gather_rows(tmain, tail, rowtail, idx, B, nchunk):
    mesh = plsc.VectorSubcoreMesh(
        core_axis_name="c", subcore_axis_name="s",
        num_cores=NC, num_subcores=NS)
    run = pl.kernel(
        functools.partial(_gather_body, nchunk),
        out_type=jax.ShapeDtypeStruct((B, T, V), jnp.float32),
        mesh=mesh,
        scratch_types=[
            pltpu.VMEM((nchunk * TM,), jnp.int32),
            pltpu.VMEM((TM, VM), jnp.float32),
            pltpu.VMEM((TM, VM), jnp.float32),
            pltpu.VMEM((T, VT), jnp.float32),
            pltpu.VMEM((T, VT), jnp.float32),
            pltpu.VMEM((T - TM, VM), jnp.float32),
            pltpu.VMEM((T - TM, VM), jnp.float32),
            pltpu.SemaphoreType.DMA,
            pltpu.SemaphoreType.DMA,
            pltpu.SemaphoreType.DMA,
            pltpu.SemaphoreType.DMA,
            pltpu.SemaphoreType.DMA,
            pltpu.SemaphoreType.DMA,
        ],
    )
    return run(tmain, tail, rowtail, idx)


def kernel(token_ids, embed_weight, proj_weight, proj_bias):
    B, T_in = token_ids.shape
    assert T_in == T and B % NW == 0 and B % BB == 0
    nchunk = B // NW

    wt = jnp.pad(proj_weight.T, ((0, 0), (0, VP - V)))
    bias2d = jnp.pad(proj_bias.reshape(1, V), ((0, 0), (0, VP - V)))
    tmain, ttail = _make_tables(embed_weight, wt, bias2d)

    tok_i32 = token_ids.astype(jnp.int32)
    tail = _make_tail(tok_i32, ttail, B)
    rowtail = _make_rowtail(tok_i32, tmain, B)

    idx = tok_i32[:, :TM].reshape(B * TM)
    return _gather_rows(tmain, tail, rowtail, idx, B, nchunk)


# speedup vs baseline: 4.2749x; 4.2749x over previous
import functools

import jax
import jax.numpy as jnp
from jax import lax
from jax.experimental import pallas as pl
from jax.experimental.pallas import tpu as pltpu
from jax.experimental.pallas import tpu_sc as plsc

V = 1000
VM = 896
VT = V - VM
VP = 1024
T = 50
TM = 48
BB = 8
NC = 2
NS = 16
NW = NC * NS


def _table_body(e_ref, wt_ref, b_ref, main_ref, tail_ref):
    res = (
        jnp.dot(e_ref[...], wt_ref[...],
                preferred_element_type=jnp.float32,
                precision=lax.Precision.HIGHEST)
        + b_ref[...]
    )
    main_ref[...] = res[:, :VM]
    tail_ref[...] = res[:, VM:V]


def _make_tables(embed_weight, wt, bias2d):
    return pl.pallas_call(
        _table_body,
        out_shape=(
            jax.ShapeDtypeStruct((V, VM), jnp.float32),
            jax.ShapeDtypeStruct((V, VT), jnp.float32),
        ),
    )(embed_weight, wt, bias2d)


def _edge_body(tok_ref, tab_ref, out_ref):
    n = tok_ref.shape[0]
    onehot = (tok_ref[...] == lax.broadcasted_iota(jnp.int32, (n, V), 1)
              ).astype(jnp.bfloat16)
    out_ref[...] = jnp.dot(onehot, tab_ref[...].astype(jnp.bfloat16),
                           preferred_element_type=jnp.float32)


def _make_edge(tok2d, table, rows_per_block):
    n, _ = tok2d.shape
    w = table.shape[1]
    return pl.pallas_call(
        _edge_body,
        grid=(n // rows_per_block,),
        in_specs=[
            pl.BlockSpec((rows_per_block, 1), lambda i: (i, 0)),
            pl.BlockSpec((V, w), lambda i: (0, 0)),
        ],
        out_specs=pl.BlockSpec((rows_per_block, w), lambda i: (i, 0)),
        out_shape=jax.ShapeDtypeStruct((n, w), jnp.float32),
    )(tok2d, table)


def _gather_body(nchunk, tmain_hbm, tail_hbm, rowtail_hbm, idx_hbm, out_hbm,
                 idx_v, rm0, rm1, bt0, bt1, br0, br1,
                 semm0, semm1, semt0, semt1, semr0, semr1):
    wid = lax.axis_index("s") * NC + lax.axis_index("c")

    pltpu.sync_copy(idx_hbm.at[pl.ds(wid * nchunk * TM, nchunk * TM)],
                    idx_v)

    def fire(j, rm, bt, br, semm, semt, semr):
        b = wid * nchunk + j
        ids = idx_v.at[pl.ds(j * TM, TM)]
        pltpu.async_copy(tmain_hbm.at[ids], rm, semm)
        pltpu.async_copy(tail_hbm.at[b], bt, semt)
        pltpu.async_copy(rowtail_hbm.at[b], br, semr)

    def drain_write(j, rm, bt, br, semm, semt, semr):
        ids = idx_v.at[pl.ds(0, TM)]
        b = wid * nchunk + j
        pltpu.make_async_copy(tmain_hbm.at[ids], rm, semm).wait()
        pltpu.sync_copy(rm, out_hbm.at[b, pl.ds(0, TM), pl.ds(0, VM)])
        pltpu.make_async_copy(tail_hbm.at[b], bt, semt).wait()
        pltpu.sync_copy(bt, out_hbm.at[b, :, pl.ds(VM, VT)])
        pltpu.make_async_copy(rowtail_hbm.at[b], br, semr).wait()
        pltpu.sync_copy(br, out_hbm.at[b, pl.ds(TM, T - TM), pl.ds(0, VM)])

    fire(0, rm0, bt0, br0, semm0, semt0, semr0)
    fire(1, rm1, bt1, br1, semm1, semt1, semr1)

    def pair_step(p, carry):
        j = 2 * p
        drain_write(j, rm0, bt0, br0, semm0, semt0, semr0)
        fire(j + 2, rm0, bt0, br0, semm0, semt0, semr0)
        drain_write(j + 1, rm1, bt1, br1, semm1, semt1, semr1)
        fire(j + 3, rm1, bt1, br1, semm1, semt1, semr1)
        return carry

    lax.fori_loop(0, nchunk // 2 - 1, pair_step, 0)
    drain_write(nchunk - 2, rm0, bt0, br0, semm0, semt0, semr0)
    drain_write(nchunk - 1, rm1, bt1, br1, semm1, semt1, semr1)


def _gather_rows(tmain, tail, rowtail, idx, B, nchunk):
    mesh = plsc.VectorSubcoreMesh(
        core_axis_name="c", subcore_axis_name="s",
        num_cores=NC, num_subcores=NS)
    run = pl.kernel(
        functools.partial(_gather_body, nchunk),
        out_type=jax.ShapeDtypeStruct((B, T, V), jnp.float32),
        mesh=mesh,
        scratch_types=[
            pltpu.VMEM((nchunk * TM,), jnp.int32),
            pltpu.VMEM((TM, VM), jnp.float32),
            pltpu.VMEM((TM, VM), jnp.float32),
            pltpu.VMEM((T, VT), jnp.float32),
            pltpu.VMEM((T, VT), jnp.float32),
            pltpu.VMEM((T - TM, VM), jnp.float32),
            pltpu.VMEM((T - TM, VM), jnp.float32),
            pltpu.SemaphoreType.DMA,
            pltpu.SemaphoreType.DMA,
            pltpu.SemaphoreType.DMA,
            pltpu.SemaphoreType.DMA,
            pltpu.SemaphoreType.DMA,
            pltpu.SemaphoreType.DMA,
        ],
    )
    return run(tmain, tail, rowtail, idx)


def kernel(token_ids, embed_weight, proj_weight, proj_bias):
    B, T_in = token_ids.shape
    assert T_in == T and B % NW == 0 and B % BB == 0
    nchunk = B // NW

    wt = jnp.pad(proj_weight.T, ((0, 0), (0, VP - V)))
    bias2d = jnp.pad(proj_bias.reshape(1, V), ((0, 0), (0, VP - V)))
    tmain, ttail = _make_tables(embed_weight, wt, bias2d)

    tok_i32 = token_ids.astype(jnp.int32)
    tail = _make_edge(tok_i32.reshape(B * T, 1), ttail,
                      rows_per_block=1600).reshape(B, T, VT)
    rowtail = _make_edge(tok_i32[:, TM:].reshape(B * (T - TM), 1), tmain,
                         rows_per_block=512).reshape(B, T - TM, VM)

    idx = tok_i32[:, :TM].reshape(B * TM)
    return _gather_rows(tmain, tail, rowtail, idx, B, nchunk)


# speedup vs baseline: 4.8986x; 1.1459x over previous
import functools

import jax
import jax.numpy as jnp
from jax import lax
from jax.experimental import pallas as pl
from jax.experimental.pallas import tpu as pltpu
from jax.experimental.pallas import tpu_sc as plsc

V = 1000
VM = 896
VT = V - VM
VP = 1024
T = 50
TM = 48
BB = 8
NC = 2
NS = 16
NW = NC * NS


def _table_body(e_ref, wt_ref, b_ref, main_ref, tail_ref):
    res = (
        jnp.dot(e_ref[...], wt_ref[...],
                preferred_element_type=jnp.float32,
                precision=lax.Precision.HIGHEST)
        + b_ref[...]
    )
    main_ref[...] = res[:, :VM]
    tail_ref[...] = res[:, VM:V]


def _make_tables(embed_weight, wt, bias2d):
    return pl.pallas_call(
        _table_body,
        out_shape=(
            jax.ShapeDtypeStruct((V, VM), jnp.float32),
            jax.ShapeDtypeStruct((V, VT), jnp.float32),
        ),
    )(embed_weight, wt, bias2d)


def _edge_body(tok_ref, tab_ref, out_ref):
    n = tok_ref.shape[0]
    onehot = (tok_ref[...] == lax.broadcasted_iota(jnp.int32, (n, V), 1)
              ).astype(jnp.bfloat16)
    out_ref[...] = jnp.dot(onehot, tab_ref[...].astype(jnp.bfloat16),
                           preferred_element_type=jnp.float32)


def _make_edge(tok2d, table, rows_per_block):
    n, _ = tok2d.shape
    w = table.shape[1]
    return pl.pallas_call(
        _edge_body,
        grid=(n // rows_per_block,),
        in_specs=[
            pl.BlockSpec((rows_per_block, 1), lambda i: (i, 0)),
            pl.BlockSpec((V, w), lambda i: (0, 0)),
        ],
        out_specs=pl.BlockSpec((rows_per_block, w), lambda i: (i, 0)),
        out_shape=jax.ShapeDtypeStruct((n, w), jnp.float32),
    )(tok2d, table)


def _gather_body(nchunk, tmain_hbm, tail_hbm, rowtail_hbm, idx_hbm, out_hbm,
                 idx_v, rm0, rm1, bt0, bt1, br0, br1,
                 semm0, semm1, semt0, semt1, semr0, semr1):
    wid = lax.axis_index("s") * NC + lax.axis_index("c")

    pltpu.sync_copy(idx_hbm.at[pl.ds(wid * nchunk * TM, nchunk * TM)],
                    idx_v)

    def fire(j, rm, bt, br, semm, semt, semr):
        b = wid * nchunk + j
        ids = idx_v.at[pl.ds(j * TM, TM)]
        pltpu.async_copy(tmain_hbm.at[ids], rm, semm)

    def drain_write(j, rm, bt, br, semm, semt, semr):
        ids = idx_v.at[pl.ds(0, TM)]
        b = wid * nchunk + j
        pltpu.make_async_copy(tmain_hbm.at[ids], rm, semm).wait()
        pltpu.sync_copy(rm, out_hbm.at[b, pl.ds(0, TM), pl.ds(0, VM)])

    fire(0, rm0, bt0, br0, semm0, semt0, semr0)
    fire(1, rm1, bt1, br1, semm1, semt1, semr1)

    def pair_step(p, carry):
        j = 2 * p
        drain_write(j, rm0, bt0, br0, semm0, semt0, semr0)
        fire(j + 2, rm0, bt0, br0, semm0, semt0, semr0)
        drain_write(j + 1, rm1, bt1, br1, semm1, semt1, semr1)
        fire(j + 3, rm1, bt1, br1, semm1, semt1, semr1)
        return carry

    lax.fori_loop(0, nchunk // 2 - 1, pair_step, 0)
    drain_write(nchunk - 2, rm0, bt0, br0, semm0, semt0, semr0)
    drain_write(nchunk - 1, rm1, bt1, br1, semm1, semt1, semr1)


def _gather_rows(tmain, tail, rowtail, idx, B, nchunk):
    mesh = plsc.VectorSubcoreMesh(
        core_axis_name="c", subcore_axis_name="s",
        num_cores=NC, num_subcores=NS)
    run = pl.kernel(
        functools.partial(_gather_body, nchunk),
        out_type=jax.ShapeDtypeStruct((B, T, V), jnp.float32),
        mesh=mesh,
        scratch_types=[
            pltpu.VMEM((nchunk * TM,), jnp.int32),
            pltpu.VMEM((TM, VM), jnp.float32),
            pltpu.VMEM((TM, VM), jnp.float32),
            pltpu.VMEM((T, VT), jnp.float32),
            pltpu.VMEM((T, VT), jnp.float32),
            pltpu.VMEM((T - TM, VM), jnp.float32),
            pltpu.VMEM((T - TM, VM), jnp.float32),
            pltpu.SemaphoreType.DMA,
            pltpu.SemaphoreType.DMA,
            pltpu.SemaphoreType.DMA,
            pltpu.SemaphoreType.DMA,
            pltpu.SemaphoreType.DMA,
            pltpu.SemaphoreType.DMA,
        ],
    )
    return run(tmain, tail, rowtail, idx)


def kernel(token_ids, embed_weight, proj_weight, proj_bias):
    B, T_in = token_ids.shape
    assert T_in == T and B % NW == 0 and B % BB == 0
    nchunk = B // NW

    wt = jnp.pad(proj_weight.T, ((0, 0), (0, VP - V)))
    bias2d = jnp.pad(proj_bias.reshape(1, V), ((0, 0), (0, VP - V)))
    tmain, ttail = _make_tables(embed_weight, wt, bias2d)

    tok_i32 = token_ids.astype(jnp.int32)
    tail = _make_edge(tok_i32.reshape(B * T, 1), ttail,
                      rows_per_block=1600).reshape(B, T, VT)
    rowtail = _make_edge(tok_i32[:, TM:].reshape(B * (T - TM), 1), tmain,
                         rows_per_block=512).reshape(B, T - TM, VM)

    idx = tok_i32[:, :TM].reshape(B * TM)
    out = _gather_rows(tmain, tail, rowtail, idx, B, nchunk)
    out = lax.dynamic_update_slice(out, tail, (0, 0, VM))
    out = lax.dynamic_update_slice(out, rowtail, (0, TM, 0))
    return out


# speedup vs baseline: 5.5424x; 1.1314x over previous
import functools

import jax
import jax.numpy as jnp
from jax import lax
from jax.experimental import pallas as pl
from jax.experimental.pallas import tpu as pltpu
from jax.experimental.pallas import tpu_sc as plsc

V = 1000
VM = 896
VT = V - VM
VP = 1024
T = 50
TM = 48
BB = 8
NC = 2
NS = 16
NW = NC * NS


def _table_body(e_ref, wt_ref, b_ref, main_ref, tail_ref):
    res = (
        jnp.dot(e_ref[...], wt_ref[...],
                preferred_element_type=jnp.float32,
                precision=lax.Precision.HIGHEST)
        + b_ref[...]
    )
    main_ref[...] = res[:, :VM]
    tail_ref[...] = res[:, VM:V]


def _make_tables(embed_weight, wt, bias2d):
    return pl.pallas_call(
        _table_body,
        out_shape=(
            jax.ShapeDtypeStruct((V, VM), jnp.float32),
            jax.ShapeDtypeStruct((V, VT), jnp.float32),
        ),
    )(embed_weight, wt, bias2d)


def _edge_body(tok_ref, tab_ref, out_ref):
    n = tok_ref.shape[0]
    onehot = (tok_ref[...] == lax.broadcasted_iota(jnp.int32, (n, V), 1)
              ).astype(jnp.bfloat16)
    out_ref[...] = jnp.dot(onehot, tab_ref[...].astype(jnp.bfloat16),
                           preferred_element_type=jnp.float32)


def _make_edge(tok2d, table, rows_per_block):
    n, _ = tok2d.shape
    w = table.shape[1]
    return pl.pallas_call(
        _edge_body,
        grid=(n // rows_per_block,),
        in_specs=[
            pl.BlockSpec((rows_per_block, 1), lambda i: (i, 0)),
            pl.BlockSpec((V, w), lambda i: (0, 0)),
        ],
        out_specs=pl.BlockSpec((rows_per_block, w), lambda i: (i, 0)),
        out_shape=jax.ShapeDtypeStruct((n, w), jnp.float32),
    )(tok2d, table)


def _gather_body(nchunk, tmain_hbm, idx_hbm, out_hbm,
                 idx_v, rm0, rm1, semm0, semm1):
    wid = lax.axis_index("s") * NC + lax.axis_index("c")

    pltpu.sync_copy(idx_hbm.at[pl.ds(wid * nchunk * TM, nchunk * TM)],
                    idx_v)

    def fire(j, rm, semm):
        ids = idx_v.at[pl.ds(j * TM, TM)]
        pltpu.async_copy(tmain_hbm.at[ids], rm, semm)

    def drain_write(j, rm, semm):
        ids = idx_v.at[pl.ds(0, TM)]
        b = wid * nchunk + j
        pltpu.make_async_copy(tmain_hbm.at[ids], rm, semm).wait()
        pltpu.sync_copy(rm, out_hbm.at[b, pl.ds(0, TM), pl.ds(0, VM)])

    fire(0, rm0, semm0)
    fire(1, rm1, semm1)

    def pair_step(p, carry):
        j = 2 * p
        drain_write(j, rm0, semm0)
        fire(j + 2, rm0, semm0)
        drain_write(j + 1, rm1, semm1)
        fire(j + 3, rm1, semm1)
        return carry

    lax.fori_loop(0, nchunk // 2 - 1, pair_step, 0)
    drain_write(nchunk - 2, rm0, semm0)
    drain_write(nchunk - 1, rm1, semm1)


def _gather_rows(tmain, idx, B, nchunk):
    mesh = plsc.VectorSubcoreMesh(
        core_axis_name="c", subcore_axis_name="s",
        num_cores=NC, num_subcores=NS)
    run = pl.kernel(
        functools.partial(_gather_body, nchunk),
        out_type=jax.ShapeDtypeStruct((B, T, V), jnp.float32),
        mesh=mesh,
        scratch_types=[
            pltpu.VMEM((nchunk * TM,), jnp.int32),
            pltpu.VMEM((TM, VM), jnp.float32),
            pltpu.VMEM((TM, VM), jnp.float32),
            pltpu.SemaphoreType.DMA,
            pltpu.SemaphoreType.DMA,
        ],
    )
    return run(tmain, idx)


def kernel(token_ids, embed_weight, proj_weight, proj_bias):
    B, T_in = token_ids.shape
    assert T_in == T and B % NW == 0 and B % BB == 0
    nchunk = B // NW

    wt = jnp.pad(proj_weight.T, ((0, 0), (0, VP - V)))
    bias2d = jnp.pad(proj_bias.reshape(1, V), ((0, 0), (0, VP - V)))
    tmain, ttail = _make_tables(embed_weight, wt, bias2d)

    tok_i32 = token_ids.astype(jnp.int32)
    tail = _make_edge(tok_i32.reshape(B * T, 1), ttail,
                      rows_per_block=1600).reshape(B, T, VT)
    rowtail = _make_edge(tok_i32[:, TM:].reshape(B * (T - TM), 1), tmain,
                         rows_per_block=512).reshape(B, T - TM, VM)

    idx = tok_i32[:, :TM].reshape(B * TM)
    out = _gather_rows(tmain, idx, B, nchunk)
    out = lax.dynamic_update_slice(out, tail, (0, 0, VM))
    out = lax.dynamic_update_slice(out, rowtail, (0, TM, 0))
    return out
